# baseline (speedup 1.0000x reference)
import functools

import jax
import jax.numpy as jnp
from jax import lax
from jax.experimental import pallas as pl
from jax.experimental.pallas import tpu as pltpu
from jax.experimental.pallas import tpu_sc as plsc

N = 10000
E = 320000
D = 128
H = 8
C = 16
DE = 16

NC = 2
NS = 16
NW = NC * NS
EPW = E // NW
KC = 128
NCH = EPW // KC
TAIL = EPW - NCH * KC
ACCN = 10240
NR = ACCN // NS
NB = 64
NH = NR // NB

_mesh = lambda: plsc.VectorSubcoreMesh(
    core_axis_name="c", subcore_axis_name="s", num_cores=NC, num_subcores=NS)
_SC_PARAMS = pltpu.CompilerParams(use_tc_tiling_on_sc=False)


def _sc_gather2(xl, xr, src, dst):
    @functools.partial(
        pl.kernel,
        out_type=(jax.ShapeDtypeStruct((E, D), jnp.float32),
                  jax.ShapeDtypeStruct((E, D), jnp.float32)),
        mesh=_mesh(),
        compiler_params=_SC_PARAMS,
        scratch_types=[
            pltpu.VMEM((KC,), jnp.int32),
            pltpu.VMEM((KC,), jnp.int32),
            pltpu.VMEM((KC,), jnp.int32),
            pltpu.VMEM((KC,), jnp.int32),
            pltpu.VMEM((KC, D), jnp.float32),
            pltpu.VMEM((KC, D), jnp.float32),
            pltpu.VMEM((KC, D), jnp.float32),
            pltpu.VMEM((KC, D), jnp.float32),
            pltpu.VMEM((TAIL,), jnp.int32),
            pltpu.VMEM((TAIL,), jnp.int32),
            pltpu.VMEM((TAIL, D), jnp.float32),
            pltpu.VMEM((TAIL, D), jnp.float32),
            pltpu.SemaphoreType.DMA,
            pltpu.SemaphoreType.DMA,
            pltpu.SemaphoreType.DMA,
            pltpu.SemaphoreType.DMA,
        ],
    )
    def k(xl_hbm, xr_hbm, src_hbm, dst_hbm, gxl_hbm, gxr_hbm,
          si0, si1, di0, di1, rl0, rl1, rr0, rr1,
          st_v, dt_v, rlt, rrt, s0l, s0r, s1l, s1r):
        wid = lax.axis_index("s") * NC + lax.axis_index("c")
        base = wid * EPW
        slots = ((si0, di0, rl0, rr0, s0l, s0r),
                 (si1, di1, rl1, rr1, s1l, s1r))

        def load_start(c, si, di, rl, rr, sl, sr):
            off = base + c * KC
            pltpu.sync_copy(src_hbm.at[pl.ds(off, KC)], si)
            pltpu.sync_copy(dst_hbm.at[pl.ds(off, KC)], di)
            pltpu.async_copy(xl_hbm.at[si], rl, sl)
            pltpu.async_copy(xr_hbm.at[di], rr, sr)

        for s in range(2):
            load_start(s, *slots[s])

        def body(g, carry):
            for s in range(2):
                si, di, rl, rr, sl, sr = slots[s]
                c = 2 * g + s
                off = base + c * KC
                pltpu.make_async_copy(xl_hbm.at[si], rl, sl).wait()
                pltpu.make_async_copy(xr_hbm.at[di], rr, sr).wait()
                pltpu.sync_copy(rl, gxl_hbm.at[pl.ds(off, KC)])
                pltpu.sync_copy(rr, gxr_hbm.at[pl.ds(off, KC)])
                cn = c + 2

                @pl.when(cn < NCH)
                def _():
                    load_start(cn, si, di, rl, rr, sl, sr)

            return carry

        lax.fori_loop(0, NCH // 2, body, 0)

        toff = base + NCH * KC
        pltpu.sync_copy(src_hbm.at[pl.ds(toff, TAIL)], st_v)
        pltpu.sync_copy(dst_hbm.at[pl.ds(toff, TAIL)], dt_v)
        pltpu.async_copy(xl_hbm.at[st_v], rlt, s0l).wait()
        pltpu.async_copy(xr_hbm.at[dt_v], rrt, s0r).wait()
        pltpu.sync_copy(rlt, gxl_hbm.at[pl.ds(toff, TAIL)])
        pltpu.sync_copy(rrt, gxr_hbm.at[pl.ds(toff, TAIL)])

    return k(xl, xr, src, dst)


def _sc_scatter_add(vals, idx, gmax16=None):
    Dp = vals.shape[1]
    has_exp = gmax16 is not None

    scratch = [
        pltpu.VMEM((KC,), jnp.int32),
        pltpu.VMEM((KC,), jnp.int32),
        pltpu.VMEM((KC, Dp), jnp.float32),
        pltpu.VMEM((KC, Dp), jnp.float32),
        pltpu.VMEM((TAIL,), jnp.int32),
        pltpu.VMEM((TAIL, Dp), jnp.float32),
        pltpu.VMEM((NB, Dp), jnp.float32),
        pltpu.VMEM_SHARED((ACCN, Dp), jnp.float32),
        pltpu.SemaphoreType.DMA,
        pltpu.SemaphoreType.DMA,
        pltpu.SemaphoreType.DMA,
        pltpu.SemaphoreType.DMA,
    ]
    if has_exp:
        scratch.append(pltpu.VMEM((1, 16), jnp.float32))

    @functools.partial(
        pl.kernel,
        out_type=jax.ShapeDtypeStruct((NC * ACCN, Dp), jnp.float32),
        mesh=_mesh(),
        compiler_params=_SC_PARAMS,
        scratch_types=scratch,
    )
    def k(*refs):
        if has_exp:
            (vals_hbm, idx_hbm, gmax_hbm, out_hbm,
             idx0, idx1, rows0, rows1, idxt, rowst, zb_v, acc_sh,
             l0, l1, w0, w1, gm_v) = refs
        else:
            (vals_hbm, idx_hbm, out_hbm,
             idx0, idx1, rows0, rows1, idxt, rowst, zb_v, acc_sh,
             l0, l1, w0, w1) = refs
        cid = lax.axis_index("c")
        sid = lax.axis_index("s")
        wid = sid * NC + cid

        def zrow(i, carry):
            for cc in range(Dp // 16):
                zb_v[i, pl.ds(cc * 16, 16)] = jnp.zeros((16,), jnp.float32)
            return carry

        lax.fori_loop(0, NB, zrow, 0)
        for hop in range(NH):
            pltpu.sync_copy(zb_v, acc_sh.at[pl.ds(sid * NR + hop * NB, NB)])
        if has_exp:
            pltpu.sync_copy(gmax_hbm, gm_v)
        plsc.subcore_barrier()

        base = wid * EPW
        slots = ((idx0, rows0, l0, w0), (idx1, rows1, l1, w1))

        def load_start(c, idxv, rowsv, lsem):
            off = base + c * KC
            pltpu.async_copy(idx_hbm.at[pl.ds(off, KC)], idxv, lsem)
            pltpu.async_copy(vals_hbm.at[pl.ds(off, KC)], rowsv, lsem)

        for s in range(2):
            load_start(s, slots[s][0], slots[s][1], slots[s][2])

        def do_exp(rowsv, nrows):
            gm = gm_v[0, :]

            def erow(r, c2):
                rowsv[r, :] = jnp.exp(rowsv[r, :] - gm)
                return c2

            lax.fori_loop(0, nrows, erow, 0)

        def body(g, carry):
            for s in range(2):
                idxv, rowsv, lsem, wsem = slots[s]
                c = 2 * g + s
                off = base + c * KC
                pltpu.make_async_copy(
                    idx_hbm.at[pl.ds(off, KC)], idxv, lsem).wait()
                pltpu.make_async_copy(
                    vals_hbm.at[pl.ds(off, KC)], rowsv, lsem).wait()
                if has_exp:
                    do_exp(rowsv, KC)
                pltpu.async_copy(rowsv, acc_sh.at[idxv], wsem, add=True)
                cn = c + 2

                @pl.when(cn < NCH)
                def _():
                    pltpu.make_async_copy(rowsv, acc_sh.at[idxv], wsem).wait()
                    load_start(cn, idxv, rowsv, lsem)

            return carry

        lax.fori_loop(0, NCH // 2, body, 0)
        for s in range(2):
            idxv, rowsv, _, wsem = slots[s]
            pltpu.make_async_copy(rowsv, acc_sh.at[idxv], wsem).wait()

        toff = base + NCH * KC
        pltpu.sync_copy(idx_hbm.at[pl.ds(toff, TAIL)], idxt)
        pltpu.sync_copy(vals_hbm.at[pl.ds(toff, TAIL)], rowst)
        if has_exp:
            do_exp(rowst, TAIL)
        pltpu.sync_copy(rowst, acc_sh.at[idxt], add=True)
        plsc.subcore_barrier()

        for hop in range(NH):
            r0 = sid * NR + hop * NB
            pltpu.sync_copy(acc_sh.at[pl.ds(r0, NB)], zb_v)
            pltpu.sync_copy(zb_v, out_hbm.at[pl.ds(cid * ACCN + r0, NB)])

    if has_exp:
        return k(vals, idx, gmax16)
    return k(vals, idx)


def _sc_msg_scatter(gxl, alpha16, gmax16, idx):
    @functools.partial(
        pl.kernel,
        out_type=jax.ShapeDtypeStruct((NC * ACCN, D), jnp.float32),
        mesh=_mesh(),
        compiler_params=_SC_PARAMS,
        scratch_types=[
            pltpu.VMEM((KC,), jnp.int32),
            pltpu.VMEM((KC,), jnp.int32),
            pltpu.VMEM((KC, D), jnp.float32),
            pltpu.VMEM((KC, D), jnp.float32),
            pltpu.VMEM((KC, 16), jnp.float32),
            pltpu.VMEM((KC, 16), jnp.float32),
            pltpu.VMEM((TAIL,), jnp.int32),
            pltpu.VMEM((TAIL, D), jnp.float32),
            pltpu.VMEM((TAIL, 16), jnp.float32),
            pltpu.VMEM((NB, D), jnp.float32),
            pltpu.VMEM_SHARED((ACCN, D), jnp.float32),
            pltpu.VMEM((1, 16), jnp.float32),
            pltpu.SemaphoreType.DMA,
            pltpu.SemaphoreType.DMA,
            pltpu.SemaphoreType.DMA,
            pltpu.SemaphoreType.DMA,
        ],
    )
    def k(gxl_hbm, al_hbm, gmax_hbm, idx_hbm, out_hbm,
          idx0, idx1, rows0, rows1, ar0, ar1, idxt, rowst, art,
          zb_v, acc_sh, gm_v, l0, l1, w0, w1):
        cid = lax.axis_index("c")
        sid = lax.axis_index("s")
        wid = sid * NC + cid

        def zrow(i, carry):
            for cc in range(D // 16):
                zb_v[i, pl.ds(cc * 16, 16)] = jnp.zeros((16,), jnp.float32)
            return carry

        lax.fori_loop(0, NB, zrow, 0)
        for hop in range(NH):
            pltpu.sync_copy(zb_v, acc_sh.at[pl.ds(sid * NR + hop * NB, NB)])
        pltpu.sync_copy(gmax_hbm, gm_v)
        plsc.subcore_barrier()

        base = wid * EPW
        slots = ((idx0, rows0, ar0, l0, w0), (idx1, rows1, ar1, l1, w1))

        def load_start(c, idxv, rowsv, arv, lsem):
            off = base + c * KC
            pltpu.async_copy(idx_hbm.at[pl.ds(off, KC)], idxv, lsem)
            pltpu.async_copy(gxl_hbm.at[pl.ds(off, KC)], rowsv, lsem)
            pltpu.async_copy(al_hbm.at[pl.ds(off, KC)], arv, lsem)

        for s in range(2):
            load_start(s, slots[s][0], slots[s][1], slots[s][2], slots[s][3])

        def scale(rowsv, arv, nrows):
            gm = gm_v[0, :]

            def erow(r, c2):
                av = jnp.exp(arv[r, :] - gm)
                for h in range(H):
                    sc = av[h]
                    rowsv[r, pl.ds(C * h, C)] = rowsv[r, pl.ds(C * h, C)] * sc
                return c2

            lax.fori_loop(0, nrows, erow, 0)

        def body(g, carry):
            for s in range(2):
                idxv, rowsv, arv, lsem, wsem = slots[s]
                c = 2 * g + s
                off = base + c * KC
                pltpu.make_async_copy(
                    idx_hbm.at[pl.ds(off, KC)], idxv, lsem).wait()
                pltpu.make_async_copy(
                    gxl_hbm.at[pl.ds(off, KC)], rowsv, lsem).wait()
                pltpu.make_async_copy(
                    al_hbm.at[pl.ds(off, KC)], arv, lsem).wait()
                scale(rowsv, arv, KC)
                pltpu.async_copy(rowsv, acc_sh.at[idxv], wsem, add=True)
                cn = c + 2

                @pl.when(cn < NCH)
                def _():
                    pltpu.make_async_copy(rowsv, acc_sh.at[idxv], wsem).wait()
                    load_start(cn, idxv, rowsv, arv, lsem)

            return carry

        lax.fori_loop(0, NCH // 2, body, 0)
        for s in range(2):
            idxv, rowsv, _, _, wsem = slots[s]
            pltpu.make_async_copy(rowsv, acc_sh.at[idxv], wsem).wait()

        toff = base + NCH * KC
        pltpu.sync_copy(idx_hbm.at[pl.ds(toff, TAIL)], idxt)
        pltpu.sync_copy(gxl_hbm.at[pl.ds(toff, TAIL)], rowst)
        pltpu.sync_copy(al_hbm.at[pl.ds(toff, TAIL)], art)
        scale(rowst, art, TAIL)
        pltpu.sync_copy(rowst, acc_sh.at[idxt], add=True)
        plsc.subcore_barrier()

        for hop in range(NH):
            r0 = sid * NR + hop * NB
            pltpu.sync_copy(acc_sh.at[pl.ds(r0, NB)], zb_v)
            pltpu.sync_copy(zb_v, out_hbm.at[pl.ds(cid * ACCN + r0, NB)])

    return k(gxl, alpha16, gmax16, idx)


def _node_proj(x, Wl, bl2, Wr, br2):
    NBLK = 2000

    def body(x_ref, wl_ref, bl_ref, wr_ref, br_ref, xl_ref, xr_ref):
        xb = x_ref[...]
        xl_ref[...] = jnp.dot(xb, wl_ref[...],
                              preferred_element_type=jnp.float32) + bl_ref[...]
        xr_ref[...] = jnp.dot(xb, wr_ref[...],
                              preferred_element_type=jnp.float32) + br_ref[...]

    return pl.pallas_call(
        body,
        grid=(N // NBLK,),
        in_specs=[
            pl.BlockSpec((NBLK, D), lambda i: (i, 0)),
            pl.BlockSpec((D, D), lambda i: (0, 0)),
            pl.BlockSpec((1, D), lambda i: (0, 0)),
            pl.BlockSpec((D, D), lambda i: (0, 0)),
            pl.BlockSpec((1, D), lambda i: (0, 0)),
        ],
        out_specs=[
            pl.BlockSpec((NBLK, D), lambda i: (i, 0)),
            pl.BlockSpec((NBLK, D), lambda i: (i, 0)),
        ],
        out_shape=[
            jax.ShapeDtypeStruct((N, D), jnp.float32),
            jax.ShapeDtypeStruct((N, D), jnp.float32),
        ],
    )(x, Wl, bl2, Wr, br2)


def _alpha(gxl, gxr, edge_attr, We, att128, sel16):
    EB = 2000

    def body(gxl_ref, gxr_ref, ea_ref, we_ref, att_ref, sel_ref,
             alpha_ref, gmax_ref):
        eab = jnp.dot(ea_ref[...], we_ref[...],
                      preferred_element_type=jnp.float32)
        m = gxl_ref[...] + gxr_ref[...] + eab
        m = jnp.where(m >= 0.0, m, 0.2 * m)
        t = m * att_ref[...]
        ab = jnp.dot(t, sel_ref[...], preferred_element_type=jnp.float32)
        alpha_ref[...] = ab
        bm = jnp.max(ab, axis=0, keepdims=True)

        @pl.when(pl.program_id(0) == 0)
        def _():
            gmax_ref[...] = bm

        @pl.when(pl.program_id(0) != 0)
        def _():
            gmax_ref[...] = jnp.maximum(gmax_ref[...], bm)

    return pl.pallas_call(
        body,
        grid=(E // EB,),
        in_specs=[
            pl.BlockSpec((EB, D), lambda i: (i, 0)),
            pl.BlockSpec((EB, D), lambda i: (i, 0)),
            pl.BlockSpec((EB, DE), lambda i: (i, 0)),
            pl.BlockSpec((DE, D), lambda i: (0, 0)),
            pl.BlockSpec((1, D), lambda i: (0, 0)),
            pl.BlockSpec((D, 2 * H), lambda i: (0, 0)),
        ],
        out_specs=[
            pl.BlockSpec((EB, 2 * H), lambda i: (i, 0)),
            pl.BlockSpec((1, 2 * H), lambda i: (0, 0)),
        ],
        out_shape=[
            jax.ShapeDtypeStruct((E, 2 * H), jnp.float32),
            jax.ShapeDtypeStruct((1, 2 * H), jnp.float32),
        ],
    )(gxl, gxr, edge_attr, We, att128, sel16)


def _wmsg(gxl, alpha16, gmax16, selT):
    EB = 2000

    def body(gxl_ref, al_ref, gm_ref, selT_ref, o_ref):
        a8 = jnp.exp(al_ref[:, :H] - gm_ref[:, :H])
        a128 = jnp.dot(a8, selT_ref[...], preferred_element_type=jnp.float32)
        o_ref[...] = gxl_ref[...] * a128

    return pl.pallas_call(
        body,
        grid=(E // EB,),
        in_specs=[
            pl.BlockSpec((EB, D), lambda i: (i, 0)),
            pl.BlockSpec((EB, 2 * H), lambda i: (i, 0)),
            pl.BlockSpec((1, 2 * H), lambda i: (0, 0)),
            pl.BlockSpec((H, D), lambda i: (0, 0)),
        ],
        out_specs=pl.BlockSpec((EB, D), lambda i: (i, 0)),
        out_shape=jax.ShapeDtypeStruct((E, D), jnp.float32),
    )(gxl, alpha16, gmax16, selT)


def _post(c2, d2, x, importance, bias2, Wg0, wgi, bg2, Wp, bp2, selT):
    NBLK = 2000

    def body(c0_ref, c1_ref, d0_ref, d1_ref, x_ref, imp_ref, bias_ref,
             wg0_ref, wgi_ref, bg_ref, wp_ref, bp_ref, selT_ref,
             out_ref, prop_ref):
        dn8 = d0_ref[:, :H] + d1_ref[:, :H]
        rec = 1.0 / (dn8 + 1e-16)
        rec128 = jnp.dot(rec, selT_ref[...], preferred_element_type=jnp.float32)
        conv = (c0_ref[...] + c1_ref[...]) * rec128 + bias_ref[...]
        logit = (jnp.dot(conv, wg0_ref[...],
                         preferred_element_type=jnp.float32)
                 + imp_ref[...] * wgi_ref[...] + bg_ref[...])
        gate = 1.0 / (1.0 + jnp.exp(-logit))
        out = gate * conv + (1.0 - gate) * x_ref[...]
        out_ref[...] = out
        prop_ref[...] = jnp.dot(out, wp_ref[...],
                                preferred_element_type=jnp.float32) + bp_ref[...]

    return pl.pallas_call(
        body,
        grid=(N // NBLK,),
        in_specs=[
            pl.BlockSpec((NBLK, D), lambda i: (i, 0)),
            pl.BlockSpec((NBLK, D), lambda i: (i, 0)),
            pl.BlockSpec((NBLK, 2 * H), lambda i: (i, 0)),
            pl.BlockSpec((NBLK, 2 * H), lambda i: (i, 0)),
            pl.BlockSpec((NBLK, D), lambda i: (i, 0)),
            pl.BlockSpec((NBLK, 1), lambda i: (i, 0)),
            pl.BlockSpec((1, D), lambda i: (0, 0)),
            pl.BlockSpec((D, D), lambda i: (0, 0)),
            pl.BlockSpec((1, D), lambda i: (0, 0)),
            pl.BlockSpec((1, D), lambda i: (0, 0)),
            pl.BlockSpec((D, 1), lambda i: (0, 0)),
            pl.BlockSpec((1, 1), lambda i: (0, 0)),
            pl.BlockSpec((H, D), lambda i: (0, 0)),
        ],
        out_specs=[
            pl.BlockSpec((NBLK, D), lambda i: (i, 0)),
            pl.BlockSpec((NBLK, 1), lambda i: (i, 0)),
        ],
        out_shape=[
            jax.ShapeDtypeStruct((N, D), jnp.float32),
            jax.ShapeDtypeStruct((N, 1), jnp.float32),
        ],
    )(c2[:N], c2[ACCN:ACCN + N], d2[:N], d2[ACCN:ACCN + N],
      x, importance, bias2, Wg0, wgi, bg2, Wp, bp2, selT)


def kernel(x, edge_index, edge_attr, importance, Wl, bl, Wr, br, We, att,
           bias, Wg, bg, Wp, bp):
    src = edge_index[0]
    dst = edge_index[1]
    att128 = att.reshape(1, H * C)
    sel = jnp.repeat(jnp.eye(H, dtype=jnp.float32), C, axis=0)
    sel16 = jnp.concatenate(
        [sel, jnp.zeros((H * C, H), jnp.float32)], axis=1)

    xl, xr = _node_proj(x, Wl, bl.reshape(1, D), Wr, br.reshape(1, D))
    gxl, gxr = _sc_gather2(xl, xr, src, dst)
    alpha16, gmax16 = _alpha(gxl, gxr, edge_attr, We, att128, sel16)
    den2 = _sc_scatter_add(alpha16, dst, gmax16)
    conv2 = _sc_msg_scatter(gxl, alpha16, gmax16, dst)
    out, prop = _post(conv2, den2, x, importance, bias.reshape(1, D),
                      Wg[:D], Wg[D].reshape(1, D), bg.reshape(1, D),
                      Wp, bp.reshape(1, 1), sel.T)
    return (out, prop)


# speedup vs baseline: 1.0397x; 1.0397x over previous
import functools

import jax
import jax.numpy as jnp
from jax import lax
from jax.experimental import pallas as pl
from jax.experimental.pallas import tpu as pltpu
from jax.experimental.pallas import tpu_sc as plsc

N = 10000
E = 320000
D = 128
H = 8
C = 16
DE = 16

NC = 2
NS = 16
NW = NC * NS
EPW = E // NW
KC = 128
NCH = EPW // KC
TAIL = EPW - NCH * KC
ACCN = 10240
NR = ACCN // NS
NB = 64
NH = NR // NB

_mesh = lambda: plsc.VectorSubcoreMesh(
    core_axis_name="c", subcore_axis_name="s", num_cores=NC, num_subcores=NS)
_SC_PARAMS = pltpu.CompilerParams(use_tc_tiling_on_sc=False)


def _sc_gather2(xl, xr, src, dst):
    @functools.partial(
        pl.kernel,
        out_type=(jax.ShapeDtypeStruct((E, D), jnp.float32),
                  jax.ShapeDtypeStruct((E, D), jnp.float32)),
        mesh=_mesh(),
        compiler_params=_SC_PARAMS,
        scratch_types=[
            pltpu.VMEM((EPW,), jnp.int32),
            pltpu.VMEM((EPW,), jnp.int32),
            pltpu.VMEM((KC, D), jnp.float32),
            pltpu.VMEM((KC, D), jnp.float32),
            pltpu.VMEM((KC, D), jnp.float32),
            pltpu.VMEM((KC, D), jnp.float32),
            pltpu.VMEM((KC, D), jnp.float32),
            pltpu.VMEM((KC, D), jnp.float32),
            pltpu.VMEM((TAIL, D), jnp.float32),
            pltpu.VMEM((TAIL, D), jnp.float32),
            pltpu.SemaphoreType.DMA,
            pltpu.SemaphoreType.DMA,
            pltpu.SemaphoreType.DMA,
            pltpu.SemaphoreType.DMA,
            pltpu.SemaphoreType.DMA,
            pltpu.SemaphoreType.DMA,
        ],
    )
    def k(xl_hbm, xr_hbm, src_hbm, dst_hbm, gxl_hbm, gxr_hbm,
          sfull, dfull, rl0, rl1, rl2, rr0, rr1, rr2, rlt, rrt,
          g0, g1, g2, w0, w1, w2):
        wid = lax.axis_index("s") * NC + lax.axis_index("c")
        base = wid * EPW
        slots = ((rl0, rr0, g0, w0), (rl1, rr1, g1, w1), (rl2, rr2, g2, w2))

        pltpu.sync_copy(src_hbm.at[pl.ds(base, EPW)], sfull)
        pltpu.sync_copy(dst_hbm.at[pl.ds(base, EPW)], dfull)

        def gather_start(c, rl, rr, gsem):
            o = c * KC
            pltpu.async_copy(xl_hbm.at[sfull.at[pl.ds(o, KC)]], rl, gsem)
            pltpu.async_copy(xr_hbm.at[dfull.at[pl.ds(o, KC)]], rr, gsem)

        def gather_wait(c, rl, rr, gsem):
            o = c * KC
            pltpu.make_async_copy(
                xl_hbm.at[sfull.at[pl.ds(o, KC)]], rl, gsem).wait()
            pltpu.make_async_copy(
                xr_hbm.at[dfull.at[pl.ds(o, KC)]], rr, gsem).wait()

        def wb_start(c, rl, rr, wsem):
            off = base + c * KC
            pltpu.async_copy(rl, gxl_hbm.at[pl.ds(off, KC)], wsem)
            pltpu.async_copy(rr, gxr_hbm.at[pl.ds(off, KC)], wsem)

        def wb_wait(c, rl, rr, wsem):
            off = base + c * KC
            pltpu.make_async_copy(rl, gxl_hbm.at[pl.ds(off, KC)], wsem).wait()
            pltpu.make_async_copy(rr, gxr_hbm.at[pl.ds(off, KC)], wsem).wait()

        for s in range(3):
            gather_start(s, slots[s][0], slots[s][1], slots[s][2])

        def body(g, carry):
            for s in range(3):
                rl, rr, gsem, wsem = slots[s]
                c = 3 * g + s
                gather_wait(c, rl, rr, gsem)
                wb_start(c, rl, rr, wsem)
                sp = (s + 2) % 3
                rlp, rrp, gsemp, wsemp = slots[sp]
                cr = c + 2

                @pl.when(jnp.logical_and(c >= 1, cr < NCH))
                def _():
                    wb_wait(c - 1, rlp, rrp, wsemp)
                    gather_start(cr, rlp, rrp, gsemp)

            return carry

        lax.fori_loop(0, NCH // 3, body, 0)
        for s in range(3):
            rl, rr, _, wsem = slots[s]
            wb_wait(NCH - 3 + s, rl, rr, wsem)

        toff = NCH * KC
        pltpu.async_copy(xl_hbm.at[sfull.at[pl.ds(toff, TAIL)]], rlt, g0)
        pltpu.async_copy(xr_hbm.at[dfull.at[pl.ds(toff, TAIL)]], rrt, g1)
        pltpu.make_async_copy(
            xl_hbm.at[sfull.at[pl.ds(toff, TAIL)]], rlt, g0).wait()
        pltpu.make_async_copy(
            xr_hbm.at[dfull.at[pl.ds(toff, TAIL)]], rrt, g1).wait()
        pltpu.sync_copy(rlt, gxl_hbm.at[pl.ds(base + toff, TAIL)])
        pltpu.sync_copy(rrt, gxr_hbm.at[pl.ds(base + toff, TAIL)])

    return k(xl, xr, src, dst)


def _sc_scatter_add(vals, idx, gmax16=None):
    Dp = vals.shape[1]
    has_exp = gmax16 is not None

    scratch = [
        pltpu.VMEM((KC,), jnp.int32),
        pltpu.VMEM((KC,), jnp.int32),
        pltpu.VMEM((KC, Dp), jnp.float32),
        pltpu.VMEM((KC, Dp), jnp.float32),
        pltpu.VMEM((TAIL,), jnp.int32),
        pltpu.VMEM((TAIL, Dp), jnp.float32),
        pltpu.VMEM((NB, Dp), jnp.float32),
        pltpu.VMEM_SHARED((ACCN, Dp), jnp.float32),
        pltpu.SemaphoreType.DMA,
        pltpu.SemaphoreType.DMA,
        pltpu.SemaphoreType.DMA,
        pltpu.SemaphoreType.DMA,
    ]
    if has_exp:
        scratch.append(pltpu.VMEM((1, 16), jnp.float32))

    @functools.partial(
        pl.kernel,
        out_type=jax.ShapeDtypeStruct((NC * ACCN, Dp), jnp.float32),
        mesh=_mesh(),
        compiler_params=_SC_PARAMS,
        scratch_types=scratch,
    )
    def k(*refs):
        if has_exp:
            (vals_hbm, idx_hbm, gmax_hbm, out_hbm,
             idx0, idx1, rows0, rows1, idxt, rowst, zb_v, acc_sh,
             l0, l1, w0, w1, gm_v) = refs
        else:
            (vals_hbm, idx_hbm, out_hbm,
             idx0, idx1, rows0, rows1, idxt, rowst, zb_v, acc_sh,
             l0, l1, w0, w1) = refs
        cid = lax.axis_index("c")
        sid = lax.axis_index("s")
        wid = sid * NC + cid

        def zrow(i, carry):
            for cc in range(Dp // 16):
                zb_v[i, pl.ds(cc * 16, 16)] = jnp.zeros((16,), jnp.float32)
            return carry

        lax.fori_loop(0, NB, zrow, 0)
        for hop in range(NH):
            pltpu.sync_copy(zb_v, acc_sh.at[pl.ds(sid * NR + hop * NB, NB)])
        if has_exp:
            pltpu.sync_copy(gmax_hbm, gm_v)
        plsc.subcore_barrier()

        base = wid * EPW
        slots = ((idx0, rows0, l0, w0), (idx1, rows1, l1, w1))

        def load_start(c, idxv, rowsv, lsem):
            off = base + c * KC
            pltpu.async_copy(idx_hbm.at[pl.ds(off, KC)], idxv, lsem)
            pltpu.async_copy(vals_hbm.at[pl.ds(off, KC)], rowsv, lsem)

        for s in range(2):
            load_start(s, slots[s][0], slots[s][1], slots[s][2])

        def do_exp(rowsv, nrows):
            gm = gm_v[0, :]

            def erow(r, c2):
                rowsv[r, :] = jnp.exp(rowsv[r, :] - gm)
                return c2

            lax.fori_loop(0, nrows, erow, 0)

        def body(g, carry):
            for s in range(2):
                idxv, rowsv, lsem, wsem = slots[s]
                c = 2 * g + s
                off = base + c * KC
                pltpu.make_async_copy(
                    idx_hbm.at[pl.ds(off, KC)], idxv, lsem).wait()
                pltpu.make_async_copy(
                    vals_hbm.at[pl.ds(off, KC)], rowsv, lsem).wait()
                if has_exp:
                    do_exp(rowsv, KC)
                pltpu.async_copy(rowsv, acc_sh.at[idxv], wsem, add=True)
                cn = c + 2

                @pl.when(cn < NCH)
                def _():
                    pltpu.make_async_copy(rowsv, acc_sh.at[idxv], wsem).wait()
                    load_start(cn, idxv, rowsv, lsem)

            return carry

        lax.fori_loop(0, NCH // 2, body, 0)
        for s in range(2):
            idxv, rowsv, _, wsem = slots[s]
            pltpu.make_async_copy(rowsv, acc_sh.at[idxv], wsem).wait()

        toff = base + NCH * KC
        pltpu.sync_copy(idx_hbm.at[pl.ds(toff, TAIL)], idxt)
        pltpu.sync_copy(vals_hbm.at[pl.ds(toff, TAIL)], rowst)
        if has_exp:
            do_exp(rowst, TAIL)
        pltpu.sync_copy(rowst, acc_sh.at[idxt], add=True)
        plsc.subcore_barrier()

        for hop in range(NH):
            r0 = sid * NR + hop * NB
            pltpu.sync_copy(acc_sh.at[pl.ds(r0, NB)], zb_v)
            pltpu.sync_copy(zb_v, out_hbm.at[pl.ds(cid * ACCN + r0, NB)])

    if has_exp:
        return k(vals, idx, gmax16)
    return k(vals, idx)


def _node_proj(x, Wl, bl2, Wr, br2):
    NBLK = 2000

    def body(x_ref, wl_ref, bl_ref, wr_ref, br_ref, xl_ref, xr_ref):
        xb = x_ref[...]
        xl_ref[...] = jnp.dot(xb, wl_ref[...],
                              preferred_element_type=jnp.float32) + bl_ref[...]
        xr_ref[...] = jnp.dot(xb, wr_ref[...],
                              preferred_element_type=jnp.float32) + br_ref[...]

    return pl.pallas_call(
        body,
        grid=(N // NBLK,),
        in_specs=[
            pl.BlockSpec((NBLK, D), lambda i: (i, 0)),
            pl.BlockSpec((D, D), lambda i: (0, 0)),
            pl.BlockSpec((1, D), lambda i: (0, 0)),
            pl.BlockSpec((D, D), lambda i: (0, 0)),
            pl.BlockSpec((1, D), lambda i: (0, 0)),
        ],
        out_specs=[
            pl.BlockSpec((NBLK, D), lambda i: (i, 0)),
            pl.BlockSpec((NBLK, D), lambda i: (i, 0)),
        ],
        out_shape=[
            jax.ShapeDtypeStruct((N, D), jnp.float32),
            jax.ShapeDtypeStruct((N, D), jnp.float32),
        ],
    )(x, Wl, bl2, Wr, br2)


def _alpha(gxl, gxr, edge_attr, We, att128, sel16):
    EB = 2000

    def body(gxl_ref, gxr_ref, ea_ref, we_ref, att_ref, sel_ref,
             alpha_ref, gmax_ref):
        eab = jnp.dot(ea_ref[...], we_ref[...],
                      preferred_element_type=jnp.float32)
        m = gxl_ref[...] + gxr_ref[...] + eab
        m = jnp.where(m >= 0.0, m, 0.2 * m)
        t = m * att_ref[...]
        ab = jnp.dot(t, sel_ref[...], preferred_element_type=jnp.float32)
        alpha_ref[...] = ab
        bm = jnp.max(ab, axis=0, keepdims=True)

        @pl.when(pl.program_id(0) == 0)
        def _():
            gmax_ref[...] = bm

        @pl.when(pl.program_id(0) != 0)
        def _():
            gmax_ref[...] = jnp.maximum(gmax_ref[...], bm)

    return pl.pallas_call(
        body,
        grid=(E // EB,),
        in_specs=[
            pl.BlockSpec((EB, D), lambda i: (i, 0)),
            pl.BlockSpec((EB, D), lambda i: (i, 0)),
            pl.BlockSpec((EB, DE), lambda i: (i, 0)),
            pl.BlockSpec((DE, D), lambda i: (0, 0)),
            pl.BlockSpec((1, D), lambda i: (0, 0)),
            pl.BlockSpec((D, 2 * H), lambda i: (0, 0)),
        ],
        out_specs=[
            pl.BlockSpec((EB, 2 * H), lambda i: (i, 0)),
            pl.BlockSpec((1, 2 * H), lambda i: (0, 0)),
        ],
        out_shape=[
            jax.ShapeDtypeStruct((E, 2 * H), jnp.float32),
            jax.ShapeDtypeStruct((1, 2 * H), jnp.float32),
        ],
    )(gxl, gxr, edge_attr, We, att128, sel16)


def _wmsg(gxl, alpha16, gmax16, selT):
    EB = 2000

    def body(gxl_ref, al_ref, gm_ref, selT_ref, o_ref):
        a8 = jnp.exp(al_ref[:, :H] - gm_ref[:, :H])
        a128 = jnp.dot(a8, selT_ref[...], preferred_element_type=jnp.float32)
        o_ref[...] = gxl_ref[...] * a128

    return pl.pallas_call(
        body,
        grid=(E // EB,),
        in_specs=[
            pl.BlockSpec((EB, D), lambda i: (i, 0)),
            pl.BlockSpec((EB, 2 * H), lambda i: (i, 0)),
            pl.BlockSpec((1, 2 * H), lambda i: (0, 0)),
            pl.BlockSpec((H, D), lambda i: (0, 0)),
        ],
        out_specs=pl.BlockSpec((EB, D), lambda i: (i, 0)),
        out_shape=jax.ShapeDtypeStruct((E, D), jnp.float32),
    )(gxl, alpha16, gmax16, selT)


def _post(c2, d2, x, importance, bias2, Wg0, wgi, bg2, Wp, bp2, selT):
    NBLK = 2000

    def body(c0_ref, c1_ref, d0_ref, d1_ref, x_ref, imp_ref, bias_ref,
             wg0_ref, wgi_ref, bg_ref, wp_ref, bp_ref, selT_ref,
             out_ref, prop_ref):
        dn8 = d0_ref[:, :H] + d1_ref[:, :H]
        rec = 1.0 / (dn8 + 1e-16)
        rec128 = jnp.dot(rec, selT_ref[...], preferred_element_type=jnp.float32)
        conv = (c0_ref[...] + c1_ref[...]) * rec128 + bias_ref[...]
        logit = (jnp.dot(conv, wg0_ref[...],
                         preferred_element_type=jnp.float32)
                 + imp_ref[...] * wgi_ref[...] + bg_ref[...])
        gate = 1.0 / (1.0 + jnp.exp(-logit))
        out = gate * conv + (1.0 - gate) * x_ref[...]
        out_ref[...] = out
        prop_ref[...] = jnp.dot(out, wp_ref[...],
                                preferred_element_type=jnp.float32) + bp_ref[...]

    return pl.pallas_call(
        body,
        grid=(N // NBLK,),
        in_specs=[
            pl.BlockSpec((NBLK, D), lambda i: (i, 0)),
            pl.BlockSpec((NBLK, D), lambda i: (i, 0)),
            pl.BlockSpec((NBLK, 2 * H), lambda i: (i, 0)),
            pl.BlockSpec((NBLK, 2 * H), lambda i: (i, 0)),
            pl.BlockSpec((NBLK, D), lambda i: (i, 0)),
            pl.BlockSpec((NBLK, 1), lambda i: (i, 0)),
            pl.BlockSpec((1, D), lambda i: (0, 0)),
            pl.BlockSpec((D, D), lambda i: (0, 0)),
            pl.BlockSpec((1, D), lambda i: (0, 0)),
            pl.BlockSpec((1, D), lambda i: (0, 0)),
            pl.BlockSpec((D, 1), lambda i: (0, 0)),
            pl.BlockSpec((1, 1), lambda i: (0, 0)),
            pl.BlockSpec((H, D), lambda i: (0, 0)),
        ],
        out_specs=[
            pl.BlockSpec((NBLK, D), lambda i: (i, 0)),
            pl.BlockSpec((NBLK, 1), lambda i: (i, 0)),
        ],
        out_shape=[
            jax.ShapeDtypeStruct((N, D), jnp.float32),
            jax.ShapeDtypeStruct((N, 1), jnp.float32),
        ],
    )(c2[:N], c2[ACCN:ACCN + N], d2[:N], d2[ACCN:ACCN + N],
      x, importance, bias2, Wg0, wgi, bg2, Wp, bp2, selT)


def kernel(x, edge_index, edge_attr, importance, Wl, bl, Wr, br, We, att,
           bias, Wg, bg, Wp, bp):
    src = edge_index[0]
    dst = edge_index[1]
    att128 = att.reshape(1, H * C)
    sel = jnp.repeat(jnp.eye(H, dtype=jnp.float32), C, axis=0)
    sel16 = jnp.concatenate(
        [sel, jnp.zeros((H * C, H), jnp.float32)], axis=1)

    xl, xr = _node_proj(x, Wl, bl.reshape(1, D), Wr, br.reshape(1, D))
    gxl, gxr = _sc_gather2(xl, xr, src, dst)
    alpha16, gmax16 = _alpha(gxl, gxr, edge_attr, We, att128, sel16)
    den2 = _sc_scatter_add(alpha16, dst, gmax16)
    wm = _wmsg(gxl, alpha16, gmax16, sel.T)
    conv2 = _sc_scatter_add(wm, dst)
    out, prop = _post(conv2, den2, x, importance, bias.reshape(1, D),
                      Wg[:D], Wg[D].reshape(1, D), bg.reshape(1, D),
                      Wp, bp.reshape(1, 1), sel.T)
    return (out, prop)


# speedup vs baseline: 1.0634x; 1.0228x over previous
import functools

import jax
import jax.numpy as jnp
from jax import lax
from jax.experimental import pallas as pl
from jax.experimental.pallas import tpu as pltpu
from jax.experimental.pallas import tpu_sc as plsc

N = 10000
E = 320000
D = 128
H = 8
C = 16
DE = 16

NC = 2
NS = 16
NW = NC * NS
EPW = E // NW
KC = 128
NCH = EPW // KC
TAIL = EPW - NCH * KC
ACCN = 10240
NR = ACCN // NS
NB = 64
NH = NR // NB

_mesh = lambda: plsc.VectorSubcoreMesh(
    core_axis_name="c", subcore_axis_name="s", num_cores=NC, num_subcores=NS)
_SC_PARAMS = pltpu.CompilerParams(use_tc_tiling_on_sc=False)


def _sc_gather2(xl, xr, src, dst):
    @functools.partial(
        pl.kernel,
        out_type=(jax.ShapeDtypeStruct((E, D), jnp.float32),
                  jax.ShapeDtypeStruct((E, D), jnp.float32)),
        mesh=_mesh(),
        compiler_params=_SC_PARAMS,
        scratch_types=[
            pltpu.VMEM((EPW,), jnp.int32),
            pltpu.VMEM((EPW,), jnp.int32),
            pltpu.VMEM((KC, D), jnp.float32),
            pltpu.VMEM((KC, D), jnp.float32),
            pltpu.VMEM((KC, D), jnp.float32),
            pltpu.VMEM((KC, D), jnp.float32),
            pltpu.VMEM((KC, D), jnp.float32),
            pltpu.VMEM((KC, D), jnp.float32),
            pltpu.VMEM((TAIL, D), jnp.float32),
            pltpu.VMEM((TAIL, D), jnp.float32),
            pltpu.SemaphoreType.DMA,
            pltpu.SemaphoreType.DMA,
            pltpu.SemaphoreType.DMA,
            pltpu.SemaphoreType.DMA,
            pltpu.SemaphoreType.DMA,
            pltpu.SemaphoreType.DMA,
        ],
    )
    def k(xl_hbm, xr_hbm, src_hbm, dst_hbm, gxl_hbm, gxr_hbm,
          sfull, dfull, rl0, rl1, rl2, rr0, rr1, rr2, rlt, rrt,
          g0, g1, g2, w0, w1, w2):
        wid = lax.axis_index("s") * NC + lax.axis_index("c")
        base = wid * EPW
        slots = ((rl0, rr0, g0, w0), (rl1, rr1, g1, w1), (rl2, rr2, g2, w2))

        pltpu.sync_copy(src_hbm.at[pl.ds(base, EPW)], sfull)
        pltpu.sync_copy(dst_hbm.at[pl.ds(base, EPW)], dfull)

        def gather_start(c, rl, rr, gsem):
            o = c * KC
            pltpu.async_copy(xl_hbm.at[sfull.at[pl.ds(o, KC)]], rl, gsem)
            pltpu.async_copy(xr_hbm.at[dfull.at[pl.ds(o, KC)]], rr, gsem)

        def gather_wait(c, rl, rr, gsem):
            o = c * KC
            pltpu.make_async_copy(
                xl_hbm.at[sfull.at[pl.ds(o, KC)]], rl, gsem).wait()
            pltpu.make_async_copy(
                xr_hbm.at[dfull.at[pl.ds(o, KC)]], rr, gsem).wait()

        def wb_start(c, rl, rr, wsem):
            off = base + c * KC
            pltpu.async_copy(rl, gxl_hbm.at[pl.ds(off, KC)], wsem)
            pltpu.async_copy(rr, gxr_hbm.at[pl.ds(off, KC)], wsem)

        def wb_wait(c, rl, rr, wsem):
            off = base + c * KC
            pltpu.make_async_copy(rl, gxl_hbm.at[pl.ds(off, KC)], wsem).wait()
            pltpu.make_async_copy(rr, gxr_hbm.at[pl.ds(off, KC)], wsem).wait()

        for s in range(3):
            gather_start(s, slots[s][0], slots[s][1], slots[s][2])

        def addrows(rl, rr, nrows):
            def arow(r, carry):
                for cc in range(D // 16):
                    sl = pl.ds(cc * 16, 16)
                    rr[r, sl] = rl[r, sl] + rr[r, sl]
                return carry

            lax.fori_loop(0, nrows, arow, 0)

        def body(g, carry):
            for s in range(3):
                rl, rr, gsem, wsem = slots[s]
                c = 3 * g + s
                gather_wait(c, rl, rr, gsem)
                addrows(rl, rr, KC)
                wb_start(c, rl, rr, wsem)
                sp = (s + 2) % 3
                rlp, rrp, gsemp, wsemp = slots[sp]
                cr = c + 2

                @pl.when(jnp.logical_and(c >= 1, cr < NCH))
                def _():
                    wb_wait(c - 1, rlp, rrp, wsemp)
                    gather_start(cr, rlp, rrp, gsemp)

            return carry

        lax.fori_loop(0, NCH // 3, body, 0)
        for s in range(3):
            rl, rr, _, wsem = slots[s]
            wb_wait(NCH - 3 + s, rl, rr, wsem)

        toff = NCH * KC
        pltpu.async_copy(xl_hbm.at[sfull.at[pl.ds(toff, TAIL)]], rlt, g0)
        pltpu.async_copy(xr_hbm.at[dfull.at[pl.ds(toff, TAIL)]], rrt, g1)
        pltpu.make_async_copy(
            xl_hbm.at[sfull.at[pl.ds(toff, TAIL)]], rlt, g0).wait()
        pltpu.make_async_copy(
            xr_hbm.at[dfull.at[pl.ds(toff, TAIL)]], rrt, g1).wait()
        addrows(rlt, rrt, TAIL)
        pltpu.sync_copy(rlt, gxl_hbm.at[pl.ds(base + toff, TAIL)])
        pltpu.sync_copy(rrt, gxr_hbm.at[pl.ds(base + toff, TAIL)])

    return k(xl, xr, src, dst)


def _sc_scatter_add(vals, idx, gmax16=None):
    Dp = vals.shape[1]
    has_exp = gmax16 is not None

    scratch = [
        pltpu.VMEM((KC,), jnp.int32),
        pltpu.VMEM((KC,), jnp.int32),
        pltpu.VMEM((KC, Dp), jnp.float32),
        pltpu.VMEM((KC, Dp), jnp.float32),
        pltpu.VMEM((TAIL,), jnp.int32),
        pltpu.VMEM((TAIL, Dp), jnp.float32),
        pltpu.VMEM((NB, Dp), jnp.float32),
        pltpu.VMEM_SHARED((ACCN, Dp), jnp.float32),
        pltpu.SemaphoreType.DMA,
        pltpu.SemaphoreType.DMA,
        pltpu.SemaphoreType.DMA,
        pltpu.SemaphoreType.DMA,
    ]
    if has_exp:
        scratch.append(pltpu.VMEM((1, 16), jnp.float32))

    @functools.partial(
        pl.kernel,
        out_type=jax.ShapeDtypeStruct((NC * ACCN, Dp), jnp.float32),
        mesh=_mesh(),
        compiler_params=_SC_PARAMS,
        scratch_types=scratch,
    )
    def k(*refs):
        if has_exp:
            (vals_hbm, idx_hbm, gmax_hbm, out_hbm,
             idx0, idx1, rows0, rows1, idxt, rowst, zb_v, acc_sh,
             l0, l1, w0, w1, gm_v) = refs
        else:
            (vals_hbm, idx_hbm, out_hbm,
             idx0, idx1, rows0, rows1, idxt, rowst, zb_v, acc_sh,
             l0, l1, w0, w1) = refs
        cid = lax.axis_index("c")
        sid = lax.axis_index("s")
        wid = sid * NC + cid

        def zrow(i, carry):
            for cc in range(Dp // 16):
                zb_v[i, pl.ds(cc * 16, 16)] = jnp.zeros((16,), jnp.float32)
            return carry

        lax.fori_loop(0, NB, zrow, 0)
        for hop in range(NH):
            pltpu.sync_copy(zb_v, acc_sh.at[pl.ds(sid * NR + hop * NB, NB)])
        if has_exp:
            pltpu.sync_copy(gmax_hbm, gm_v)
        plsc.subcore_barrier()

        base = wid * EPW
        slots = ((idx0, rows0, l0, w0), (idx1, rows1, l1, w1))

        def load_start(c, idxv, rowsv, lsem):
            off = base + c * KC
            pltpu.async_copy(idx_hbm.at[pl.ds(off, KC)], idxv, lsem)
            pltpu.async_copy(vals_hbm.at[pl.ds(off, KC)], rowsv, lsem)

        for s in range(2):
            load_start(s, slots[s][0], slots[s][1], slots[s][2])

        def do_exp(rowsv, nrows):
            gm = gm_v[0, :]

            def erow(r, c2):
                rowsv[r, :] = jnp.exp(rowsv[r, :] - gm)
                return c2

            lax.fori_loop(0, nrows, erow, 0)

        def body(g, carry):
            for s in range(2):
                idxv, rowsv, lsem, wsem = slots[s]
                c = 2 * g + s
                off = base + c * KC
                pltpu.make_async_copy(
                    idx_hbm.at[pl.ds(off, KC)], idxv, lsem).wait()
                pltpu.make_async_copy(
                    vals_hbm.at[pl.ds(off, KC)], rowsv, lsem).wait()
                if has_exp:
                    do_exp(rowsv, KC)
                pltpu.async_copy(rowsv, acc_sh.at[idxv], wsem, add=True)
                cn = c + 2

                @pl.when(cn < NCH)
                def _():
                    pltpu.make_async_copy(rowsv, acc_sh.at[idxv], wsem).wait()
                    load_start(cn, idxv, rowsv, lsem)

            return carry

        lax.fori_loop(0, NCH // 2, body, 0)
        for s in range(2):
            idxv, rowsv, _, wsem = slots[s]
            pltpu.make_async_copy(rowsv, acc_sh.at[idxv], wsem).wait()

        toff = base + NCH * KC
        pltpu.sync_copy(idx_hbm.at[pl.ds(toff, TAIL)], idxt)
        pltpu.sync_copy(vals_hbm.at[pl.ds(toff, TAIL)], rowst)
        if has_exp:
            do_exp(rowst, TAIL)
        pltpu.sync_copy(rowst, acc_sh.at[idxt], add=True)
        plsc.subcore_barrier()

        for hop in range(NH):
            r0 = sid * NR + hop * NB
            pltpu.sync_copy(acc_sh.at[pl.ds(r0, NB)], zb_v)
            pltpu.sync_copy(zb_v, out_hbm.at[pl.ds(cid * ACCN + r0, NB)])

    if has_exp:
        return k(vals, idx, gmax16)
    return k(vals, idx)


def _node_proj(x, Wl, bl2, Wr, br2):
    NBLK = 2000

    def body(x_ref, wl_ref, bl_ref, wr_ref, br_ref, xl_ref, xr_ref):
        xb = x_ref[...]
        xl_ref[...] = jnp.dot(xb, wl_ref[...],
                              preferred_element_type=jnp.float32) + bl_ref[...]
        xr_ref[...] = jnp.dot(xb, wr_ref[...],
                              preferred_element_type=jnp.float32) + br_ref[...]

    return pl.pallas_call(
        body,
        grid=(N // NBLK,),
        in_specs=[
            pl.BlockSpec((NBLK, D), lambda i: (i, 0)),
            pl.BlockSpec((D, D), lambda i: (0, 0)),
            pl.BlockSpec((1, D), lambda i: (0, 0)),
            pl.BlockSpec((D, D), lambda i: (0, 0)),
            pl.BlockSpec((1, D), lambda i: (0, 0)),
        ],
        out_specs=[
            pl.BlockSpec((NBLK, D), lambda i: (i, 0)),
            pl.BlockSpec((NBLK, D), lambda i: (i, 0)),
        ],
        out_shape=[
            jax.ShapeDtypeStruct((N, D), jnp.float32),
            jax.ShapeDtypeStruct((N, D), jnp.float32),
        ],
    )(x, Wl, bl2, Wr, br2)


def _alpha(msum, edge_attr, We, att128, sel16):
    EB = 2000

    def body(ms_ref, ea_ref, we_ref, att_ref, sel_ref,
             alpha_ref, gmax_ref):
        eab = jnp.dot(ea_ref[...], we_ref[...],
                      preferred_element_type=jnp.float32)
        m = ms_ref[...] + eab
        m = jnp.where(m >= 0.0, m, 0.2 * m)
        t = m * att_ref[...]
        ab = jnp.dot(t, sel_ref[...], preferred_element_type=jnp.float32)
        alpha_ref[...] = ab
        bm = jnp.max(ab, axis=0, keepdims=True)

        @pl.when(pl.program_id(0) == 0)
        def _():
            gmax_ref[...] = bm

        @pl.when(pl.program_id(0) != 0)
        def _():
            gmax_ref[...] = jnp.maximum(gmax_ref[...], bm)

    return pl.pallas_call(
        body,
        grid=(E // EB,),
        in_specs=[
            pl.BlockSpec((EB, D), lambda i: (i, 0)),
            pl.BlockSpec((EB, DE), lambda i: (i, 0)),
            pl.BlockSpec((DE, D), lambda i: (0, 0)),
            pl.BlockSpec((1, D), lambda i: (0, 0)),
            pl.BlockSpec((D, 2 * H), lambda i: (0, 0)),
        ],
        out_specs=[
            pl.BlockSpec((EB, 2 * H), lambda i: (i, 0)),
            pl.BlockSpec((1, 2 * H), lambda i: (0, 0)),
        ],
        out_shape=[
            jax.ShapeDtypeStruct((E, 2 * H), jnp.float32),
            jax.ShapeDtypeStruct((1, 2 * H), jnp.float32),
        ],
    )(msum, edge_attr, We, att128, sel16)


def _wmsg(gxl, alpha16, gmax16, selT):
    EB = 2000

    def body(gxl_ref, al_ref, gm_ref, selT_ref, o_ref):
        a8 = jnp.exp(al_ref[:, :H] - gm_ref[:, :H])
        a128 = jnp.dot(a8, selT_ref[...], preferred_element_type=jnp.float32)
        o_ref[...] = gxl_ref[...] * a128

    return pl.pallas_call(
        body,
        grid=(E // EB,),
        in_specs=[
            pl.BlockSpec((EB, D), lambda i: (i, 0)),
            pl.BlockSpec((EB, 2 * H), lambda i: (i, 0)),
            pl.BlockSpec((1, 2 * H), lambda i: (0, 0)),
            pl.BlockSpec((H, D), lambda i: (0, 0)),
        ],
        out_specs=pl.BlockSpec((EB, D), lambda i: (i, 0)),
        out_shape=jax.ShapeDtypeStruct((E, D), jnp.float32),
    )(gxl, alpha16, gmax16, selT)


def _post(c2, d2, x, importance, bias2, Wg0, wgi, bg2, Wp, bp2, selT):
    NBLK = 2000

    def body(c0_ref, c1_ref, d0_ref, d1_ref, x_ref, imp_ref, bias_ref,
             wg0_ref, wgi_ref, bg_ref, wp_ref, bp_ref, selT_ref,
             out_ref, prop_ref):
        dn8 = d0_ref[:, :H] + d1_ref[:, :H]
        rec = 1.0 / (dn8 + 1e-16)
        rec128 = jnp.dot(rec, selT_ref[...], preferred_element_type=jnp.float32)
        conv = (c0_ref[...] + c1_ref[...]) * rec128 + bias_ref[...]
        logit = (jnp.dot(conv, wg0_ref[...],
                         preferred_element_type=jnp.float32)
                 + imp_ref[...] * wgi_ref[...] + bg_ref[...])
        gate = 1.0 / (1.0 + jnp.exp(-logit))
        out = gate * conv + (1.0 - gate) * x_ref[...]
        out_ref[...] = out
        prop_ref[...] = jnp.dot(out, wp_ref[...],
                                preferred_element_type=jnp.float32) + bp_ref[...]

    return pl.pallas_call(
        body,
        grid=(N // NBLK,),
        in_specs=[
            pl.BlockSpec((NBLK, D), lambda i: (i, 0)),
            pl.BlockSpec((NBLK, D), lambda i: (i, 0)),
            pl.BlockSpec((NBLK, 2 * H), lambda i: (i, 0)),
            pl.BlockSpec((NBLK, 2 * H), lambda i: (i, 0)),
            pl.BlockSpec((NBLK, D), lambda i: (i, 0)),
            pl.BlockSpec((NBLK, 1), lambda i: (i, 0)),
            pl.BlockSpec((1, D), lambda i: (0, 0)),
            pl.BlockSpec((D, D), lambda i: (0, 0)),
            pl.BlockSpec((1, D), lambda i: (0, 0)),
            pl.BlockSpec((1, D), lambda i: (0, 0)),
            pl.BlockSpec((D, 1), lambda i: (0, 0)),
            pl.BlockSpec((1, 1), lambda i: (0, 0)),
            pl.BlockSpec((H, D), lambda i: (0, 0)),
        ],
        out_specs=[
            pl.BlockSpec((NBLK, D), lambda i: (i, 0)),
            pl.BlockSpec((NBLK, 1), lambda i: (i, 0)),
        ],
        out_shape=[
            jax.ShapeDtypeStruct((N, D), jnp.float32),
            jax.ShapeDtypeStruct((N, 1), jnp.float32),
        ],
    )(c2[:N], c2[ACCN:ACCN + N], d2[:N], d2[ACCN:ACCN + N],
      x, importance, bias2, Wg0, wgi, bg2, Wp, bp2, selT)


def kernel(x, edge_index, edge_attr, importance, Wl, bl, Wr, br, We, att,
           bias, Wg, bg, Wp, bp):
    src = edge_index[0]
    dst = edge_index[1]
    att128 = att.reshape(1, H * C)
    sel = jnp.repeat(jnp.eye(H, dtype=jnp.float32), C, axis=0)
    sel16 = jnp.concatenate(
        [sel, jnp.zeros((H * C, H), jnp.float32)], axis=1)

    xl, xr = _node_proj(x, Wl, bl.reshape(1, D), Wr, br.reshape(1, D))
    gxl, msum = _sc_gather2(xl, xr, src, dst)
    alpha16, gmax16 = _alpha(msum, edge_attr, We, att128, sel16)
    den2 = _sc_scatter_add(alpha16, dst, gmax16)
    wm = _wmsg(gxl, alpha16, gmax16, sel.T)
    conv2 = _sc_scatter_add(wm, dst)
    out, prop = _post(conv2, den2, x, importance, bias.reshape(1, D),
                      Wg[:D], Wg[D].reshape(1, D), bg.reshape(1, D),
                      Wp, bp.reshape(1, 1), sel.T)
    return (out, prop)


# speedup vs baseline: 1.0636x; 1.0002x over previous
import functools

import jax
import jax.numpy as jnp
from jax import lax
from jax.experimental import pallas as pl
from jax.experimental.pallas import tpu as pltpu
from jax.experimental.pallas import tpu_sc as plsc

N = 10000
E = 320000
D = 128
H = 8
C = 16
DE = 16

NC = 2
NS = 16
NW = NC * NS
EPW = E // NW
KC = 128
NCH = EPW // KC
TAIL = EPW - NCH * KC
ACCN = 10240
NR = ACCN // NS
NB = 64
NH = NR // NB

_mesh = lambda: plsc.VectorSubcoreMesh(
    core_axis_name="c", subcore_axis_name="s", num_cores=NC, num_subcores=NS)
_SC_PARAMS = pltpu.CompilerParams(use_tc_tiling_on_sc=False)


def _sc_gather2(xl, xr, src, dst):
    @functools.partial(
        pl.kernel,
        out_type=(jax.ShapeDtypeStruct((E, D), jnp.float32),
                  jax.ShapeDtypeStruct((E, D), jnp.float32)),
        mesh=_mesh(),
        compiler_params=_SC_PARAMS,
        scratch_types=[
            pltpu.VMEM((EPW,), jnp.int32),
            pltpu.VMEM((EPW,), jnp.int32),
            pltpu.VMEM((KC, D), jnp.float32),
            pltpu.VMEM((KC, D), jnp.float32),
            pltpu.VMEM((KC, D), jnp.float32),
            pltpu.VMEM((KC, D), jnp.float32),
            pltpu.VMEM((KC, D), jnp.float32),
            pltpu.VMEM((KC, D), jnp.float32),
            pltpu.VMEM((TAIL, D), jnp.float32),
            pltpu.VMEM((TAIL, D), jnp.float32),
            pltpu.SemaphoreType.DMA,
            pltpu.SemaphoreType.DMA,
            pltpu.SemaphoreType.DMA,
            pltpu.SemaphoreType.DMA,
            pltpu.SemaphoreType.DMA,
            pltpu.SemaphoreType.DMA,
        ],
    )
    def k(xl_hbm, xr_hbm, src_hbm, dst_hbm, gxl_hbm, gxr_hbm,
          sfull, dfull, rl0, rl1, rl2, rr0, rr1, rr2, rlt, rrt,
          g0, g1, g2, w0, w1, w2):
        wid = lax.axis_index("s") * NC + lax.axis_index("c")
        base = wid * EPW
        slots = ((rl0, rr0, g0, w0), (rl1, rr1, g1, w1), (rl2, rr2, g2, w2))

        pltpu.sync_copy(src_hbm.at[pl.ds(base, EPW)], sfull)
        pltpu.sync_copy(dst_hbm.at[pl.ds(base, EPW)], dfull)

        def gather_start(c, rl, rr, gsem):
            o = c * KC
            pltpu.async_copy(xl_hbm.at[sfull.at[pl.ds(o, KC)]], rl, gsem)
            pltpu.async_copy(xr_hbm.at[dfull.at[pl.ds(o, KC)]], rr, gsem)

        def gather_wait(c, rl, rr, gsem):
            o = c * KC
            pltpu.make_async_copy(
                xl_hbm.at[sfull.at[pl.ds(o, KC)]], rl, gsem).wait()
            pltpu.make_async_copy(
                xr_hbm.at[dfull.at[pl.ds(o, KC)]], rr, gsem).wait()

        def wb_start(c, rl, rr, wsem):
            off = base + c * KC
            pltpu.async_copy(rl, gxl_hbm.at[pl.ds(off, KC)], wsem)
            pltpu.async_copy(rr, gxr_hbm.at[pl.ds(off, KC)], wsem)

        def wb_wait(c, rl, rr, wsem):
            off = base + c * KC
            pltpu.make_async_copy(rl, gxl_hbm.at[pl.ds(off, KC)], wsem).wait()
            pltpu.make_async_copy(rr, gxr_hbm.at[pl.ds(off, KC)], wsem).wait()

        for s in range(3):
            gather_start(s, slots[s][0], slots[s][1], slots[s][2])

        def addrows(rl, rr, nrows):
            def arow(r, carry):
                for cc in range(D // 16):
                    sl = pl.ds(cc * 16, 16)
                    rr[r, sl] = rl[r, sl] + rr[r, sl]
                return carry

            lax.fori_loop(0, nrows, arow, 0)

        def body(g, carry):
            for s in range(3):
                rl, rr, gsem, wsem = slots[s]
                c = 3 * g + s
                gather_wait(c, rl, rr, gsem)
                addrows(rl, rr, KC)
                wb_start(c, rl, rr, wsem)
                sp = (s + 2) % 3
                rlp, rrp, gsemp, wsemp = slots[sp]
                cr = c + 2

                @pl.when(jnp.logical_and(c >= 1, cr < NCH))
                def _():
                    wb_wait(c - 1, rlp, rrp, wsemp)
                    gather_start(cr, rlp, rrp, gsemp)

            return carry

        lax.fori_loop(0, NCH // 3, body, 0)
        for s in range(3):
            rl, rr, _, wsem = slots[s]
            wb_wait(NCH - 3 + s, rl, rr, wsem)

        toff = NCH * KC
        pltpu.async_copy(xl_hbm.at[sfull.at[pl.ds(toff, TAIL)]], rlt, g0)
        pltpu.async_copy(xr_hbm.at[dfull.at[pl.ds(toff, TAIL)]], rrt, g1)
        pltpu.make_async_copy(
            xl_hbm.at[sfull.at[pl.ds(toff, TAIL)]], rlt, g0).wait()
        pltpu.make_async_copy(
            xr_hbm.at[dfull.at[pl.ds(toff, TAIL)]], rrt, g1).wait()
        addrows(rlt, rrt, TAIL)
        pltpu.sync_copy(rlt, gxl_hbm.at[pl.ds(base + toff, TAIL)])
        pltpu.sync_copy(rrt, gxr_hbm.at[pl.ds(base + toff, TAIL)])

    return k(xl, xr, src, dst)


def _sc_scatter_add(vals, idx, gmax16=None):
    Dp = vals.shape[1]
    has_exp = gmax16 is not None

    scratch = [
        pltpu.VMEM((KC,), jnp.int32),
        pltpu.VMEM((KC,), jnp.int32),
        pltpu.VMEM((KC, Dp), jnp.float32),
        pltpu.VMEM((KC, Dp), jnp.float32),
        pltpu.VMEM((TAIL,), jnp.int32),
        pltpu.VMEM((TAIL, Dp), jnp.float32),
        pltpu.VMEM((NB, Dp), jnp.float32),
        pltpu.VMEM_SHARED((ACCN, Dp), jnp.float32),
        pltpu.SemaphoreType.DMA,
        pltpu.SemaphoreType.DMA,
        pltpu.SemaphoreType.DMA,
        pltpu.SemaphoreType.DMA,
    ]
    if has_exp:
        scratch.append(pltpu.VMEM((1, 16), jnp.float32))

    @functools.partial(
        pl.kernel,
        out_type=jax.ShapeDtypeStruct((NC * ACCN, Dp), jnp.float32),
        mesh=_mesh(),
        compiler_params=_SC_PARAMS,
        scratch_types=scratch,
    )
    def k(*refs):
        if has_exp:
            (vals_hbm, idx_hbm, gmax_hbm, out_hbm,
             idx0, idx1, rows0, rows1, idxt, rowst, zb_v, acc_sh,
             l0, l1, w0, w1, gm_v) = refs
        else:
            (vals_hbm, idx_hbm, out_hbm,
             idx0, idx1, rows0, rows1, idxt, rowst, zb_v, acc_sh,
             l0, l1, w0, w1) = refs
        cid = lax.axis_index("c")
        sid = lax.axis_index("s")
        wid = sid * NC + cid

        def zrow(i, carry):
            for cc in range(Dp // 16):
                zb_v[i, pl.ds(cc * 16, 16)] = jnp.zeros((16,), jnp.float32)
            return carry

        lax.fori_loop(0, NB, zrow, 0)
        for hop in range(NH):
            pltpu.sync_copy(zb_v, acc_sh.at[pl.ds(sid * NR + hop * NB, NB)])
        if has_exp:
            pltpu.sync_copy(gmax_hbm, gm_v)
        plsc.subcore_barrier()

        base = wid * EPW
        slots = ((idx0, rows0, l0, w0), (idx1, rows1, l1, w1))

        def load_start(c, idxv, rowsv, lsem):
            off = base + c * KC
            pltpu.async_copy(idx_hbm.at[pl.ds(off, KC)], idxv, lsem)
            pltpu.async_copy(vals_hbm.at[pl.ds(off, KC)], rowsv, lsem)

        for s in range(2):
            load_start(s, slots[s][0], slots[s][1], slots[s][2])

        def do_exp(rowsv, nrows):
            gm = gm_v[0, :]

            def erow(r, c2):
                rowsv[r, :] = jnp.exp(rowsv[r, :] - gm)
                return c2

            lax.fori_loop(0, nrows, erow, 0)

        def body(g, carry):
            for s in range(2):
                idxv, rowsv, lsem, wsem = slots[s]
                c = 2 * g + s
                off = base + c * KC
                pltpu.make_async_copy(
                    idx_hbm.at[pl.ds(off, KC)], idxv, lsem).wait()
                pltpu.make_async_copy(
                    vals_hbm.at[pl.ds(off, KC)], rowsv, lsem).wait()
                if has_exp:
                    do_exp(rowsv, KC)
                pltpu.async_copy(rowsv, acc_sh.at[idxv], wsem, add=True)
                cn = c + 2

                @pl.when(cn < NCH)
                def _():
                    pltpu.make_async_copy(rowsv, acc_sh.at[idxv], wsem).wait()
                    load_start(cn, idxv, rowsv, lsem)

            return carry

        lax.fori_loop(0, NCH // 2, body, 0)
        for s in range(2):
            idxv, rowsv, _, wsem = slots[s]
            pltpu.make_async_copy(rowsv, acc_sh.at[idxv], wsem).wait()

        toff = base + NCH * KC
        pltpu.sync_copy(idx_hbm.at[pl.ds(toff, TAIL)], idxt)
        pltpu.sync_copy(vals_hbm.at[pl.ds(toff, TAIL)], rowst)
        if has_exp:
            do_exp(rowst, TAIL)
        pltpu.sync_copy(rowst, acc_sh.at[idxt], add=True)
        plsc.subcore_barrier()

        for hop in range(NH):
            r0 = sid * NR + hop * NB
            pltpu.sync_copy(acc_sh.at[pl.ds(r0, NB)], zb_v)
            pltpu.sync_copy(zb_v, out_hbm.at[pl.ds(cid * ACCN + r0, NB)])

    if has_exp:
        return k(vals, idx, gmax16)
    return k(vals, idx)


def _node_proj(x, Wl, bl2, Wr, br2):
    NBLK = 2000

    def body(x_ref, wl_ref, bl_ref, wr_ref, br_ref, xl_ref, xr_ref):
        xb = x_ref[...]
        xl_ref[...] = jnp.dot(xb, wl_ref[...],
                              preferred_element_type=jnp.float32) + bl_ref[...]
        xr_ref[...] = jnp.dot(xb, wr_ref[...],
                              preferred_element_type=jnp.float32) + br_ref[...]

    return pl.pallas_call(
        body,
        grid=(N // NBLK,),
        in_specs=[
            pl.BlockSpec((NBLK, D), lambda i: (i, 0)),
            pl.BlockSpec((D, D), lambda i: (0, 0)),
            pl.BlockSpec((1, D), lambda i: (0, 0)),
            pl.BlockSpec((D, D), lambda i: (0, 0)),
            pl.BlockSpec((1, D), lambda i: (0, 0)),
        ],
        out_specs=[
            pl.BlockSpec((NBLK, D), lambda i: (i, 0)),
            pl.BlockSpec((NBLK, D), lambda i: (i, 0)),
        ],
        out_shape=[
            jax.ShapeDtypeStruct((N, D), jnp.float32),
            jax.ShapeDtypeStruct((N, D), jnp.float32),
        ],
    )(x, Wl, bl2, Wr, br2)


def _alpha(msum, edge_attr, We, att128, sel16):
    EB = 2000

    def body(ms_ref, ea_ref, we_ref, att_ref, sel_ref,
             alpha_ref, gmax_ref):
        eab = jnp.dot(ea_ref[...], we_ref[...],
                      preferred_element_type=jnp.float32)
        m = ms_ref[...] + eab
        m = jnp.where(m >= 0.0, m, 0.2 * m)
        t = m * att_ref[...]
        ab = jnp.dot(t, sel_ref[...], preferred_element_type=jnp.float32)
        alpha_ref[...] = ab
        bm = jnp.max(ab, axis=0, keepdims=True)

        @pl.when(pl.program_id(0) == 0)
        def _():
            gmax_ref[...] = bm

        @pl.when(pl.program_id(0) != 0)
        def _():
            gmax_ref[...] = jnp.maximum(gmax_ref[...], bm)

    return pl.pallas_call(
        body,
        grid=(E // EB,),
        in_specs=[
            pl.BlockSpec((EB, D), lambda i: (i, 0)),
            pl.BlockSpec((EB, DE), lambda i: (i, 0)),
            pl.BlockSpec((DE, D), lambda i: (0, 0)),
            pl.BlockSpec((1, D), lambda i: (0, 0)),
            pl.BlockSpec((D, 2 * H), lambda i: (0, 0)),
        ],
        out_specs=[
            pl.BlockSpec((EB, 2 * H), lambda i: (i, 0)),
            pl.BlockSpec((1, 2 * H), lambda i: (0, 0)),
        ],
        out_shape=[
            jax.ShapeDtypeStruct((E, 2 * H), jnp.float32),
            jax.ShapeDtypeStruct((1, 2 * H), jnp.float32),
        ],
    )(msum, edge_attr, We, att128, sel16)


def _wmsg(gxl, alpha16, gmax16, selT):
    EB = 2000

    def body(gxl_ref, al_ref, gm_ref, selT_ref, o_ref):
        a8 = jnp.exp(al_ref[:, :H] - gm_ref[:, :H])
        a128 = jnp.dot(a8, selT_ref[...], preferred_element_type=jnp.float32)
        o_ref[...] = gxl_ref[...] * a128

    return pl.pallas_call(
        body,
        grid=(E // EB,),
        in_specs=[
            pl.BlockSpec((EB, D), lambda i: (i, 0)),
            pl.BlockSpec((EB, 2 * H), lambda i: (i, 0)),
            pl.BlockSpec((1, 2 * H), lambda i: (0, 0)),
            pl.BlockSpec((H, D), lambda i: (0, 0)),
        ],
        out_specs=pl.BlockSpec((EB, D), lambda i: (i, 0)),
        out_shape=jax.ShapeDtypeStruct((E, D), jnp.float32),
    )(gxl, alpha16, gmax16, selT)


def _post(c2, d2, x, importance, bias2, Wg0, wgi, bg2, Wp, bp2, selT):
    NBLK = 2000

    def body(c0_ref, c1_ref, d0_ref, d1_ref, x_ref, imp_ref, bias_ref,
             wg0_ref, wgi_ref, bg_ref, wp_ref, bp_ref, selT_ref,
             out_ref, prop_ref):
        dn8 = d0_ref[:, :H] + d1_ref[:, :H]
        rec = 1.0 / (dn8 + 1e-16)
        rec128 = jnp.dot(rec, selT_ref[...], preferred_element_type=jnp.float32)
        conv = (c0_ref[...] + c1_ref[...]) * rec128 + bias_ref[...]
        logit = (jnp.dot(conv, wg0_ref[...],
                         preferred_element_type=jnp.float32)
                 + imp_ref[...] * wgi_ref[...] + bg_ref[...])
        gate = 1.0 / (1.0 + jnp.exp(-logit))
        out = gate * conv + (1.0 - gate) * x_ref[...]
        out_ref[...] = out
        prop_ref[...] = jnp.dot(out, wp_ref[...],
                                preferred_element_type=jnp.float32) + bp_ref[...]

    return pl.pallas_call(
        body,
        grid=(N // NBLK,),
        in_specs=[
            pl.BlockSpec((NBLK, D), lambda i: (i, 0)),
            pl.BlockSpec((NBLK, D), lambda i: (i, 0)),
            pl.BlockSpec((NBLK, 2 * H), lambda i: (i, 0)),
            pl.BlockSpec((NBLK, 2 * H), lambda i: (i, 0)),
            pl.BlockSpec((NBLK, D), lambda i: (i, 0)),
            pl.BlockSpec((NBLK, 1), lambda i: (i, 0)),
            pl.BlockSpec((1, D), lambda i: (0, 0)),
            pl.BlockSpec((D, D), lambda i: (0, 0)),
            pl.BlockSpec((1, D), lambda i: (0, 0)),
            pl.BlockSpec((1, D), lambda i: (0, 0)),
            pl.BlockSpec((D, 1), lambda i: (0, 0)),
            pl.BlockSpec((1, 1), lambda i: (0, 0)),
            pl.BlockSpec((H, D), lambda i: (0, 0)),
        ],
        out_specs=[
            pl.BlockSpec((NBLK, D), lambda i: (i, 0)),
            pl.BlockSpec((NBLK, 1), lambda i: (i, 0)),
        ],
        out_shape=[
            jax.ShapeDtypeStruct((N, D), jnp.float32),
            jax.ShapeDtypeStruct((N, 1), jnp.float32),
        ],
    )(c2[:N], c2[ACCN:ACCN + N], d2[:N], d2[ACCN:ACCN + N],
      x, importance, bias2, Wg0, wgi, bg2, Wp, bp2, selT)


def kernel(x, edge_index, edge_attr, importance, Wl, bl, Wr, br, We, att,
           bias, Wg, bg, Wp, bp):
    src = edge_index[0]
    dst = edge_index[1]
    att128 = att.reshape(1, H * C)
    sel = jnp.repeat(jnp.eye(H, dtype=jnp.float32), C, axis=0)
    sel16 = jnp.concatenate(
        [sel, jnp.zeros((H * C, H), jnp.float32)], axis=1)

    xl, xr = _node_proj(x, Wl, bl.reshape(1, D), Wr, br.reshape(1, D))
    gxl, msum = _sc_gather2(xl, xr, src, dst)
    alpha16, gmax16 = _alpha(msum, edge_attr, We, att128, sel16)
    wm = _wmsg(gxl, alpha16, gmax16, sel.T)
    conv2 = _sc_scatter_add(wm, dst)
    den2 = _sc_scatter_add(alpha16, dst, gmax16)
    out, prop = _post(conv2, den2, x, importance, bias.reshape(1, D),
                      Wg[:D], Wg[D].reshape(1, D), bg.reshape(1, D),
                      Wp, bp.reshape(1, 1), sel.T)
    return (out, prop)


# speedup vs baseline: 1.1675x; 1.0977x over previous
import functools

import jax
import jax.numpy as jnp
from jax import lax
from jax.experimental import pallas as pl
from jax.experimental.pallas import tpu as pltpu
from jax.experimental.pallas import tpu_sc as plsc

N = 10000
E = 320000
D = 128
H = 8
C = 16
DE = 16

NC = 2
NS = 16
NW = NC * NS
EPW = E // NW
KC = 128
NCH = EPW // KC
TAIL = EPW - NCH * KC
ACCN = 10240
NR = ACCN // NS
NB = 64
NH = NR // NB

_mesh = lambda: plsc.VectorSubcoreMesh(
    core_axis_name="c", subcore_axis_name="s", num_cores=NC, num_subcores=NS)
_SC_PARAMS = pltpu.CompilerParams(use_tc_tiling_on_sc=False)


def _sc_gather2(xl, xr, src, dst):
    @functools.partial(
        pl.kernel,
        out_type=(jax.ShapeDtypeStruct((E, D), jnp.float32),
                  jax.ShapeDtypeStruct((E, D), jnp.float32)),
        mesh=_mesh(),
        compiler_params=_SC_PARAMS,
        scratch_types=[
            pltpu.VMEM((EPW,), jnp.int32),
            pltpu.VMEM((EPW,), jnp.int32),
            pltpu.VMEM((KC, D), jnp.float32),
            pltpu.VMEM((KC, D), jnp.float32),
            pltpu.VMEM((KC, D), jnp.float32),
            pltpu.VMEM((KC, D), jnp.float32),
            pltpu.VMEM((KC, D), jnp.float32),
            pltpu.VMEM((KC, D), jnp.float32),
            pltpu.VMEM((TAIL, D), jnp.float32),
            pltpu.VMEM((TAIL, D), jnp.float32),
            pltpu.SemaphoreType.DMA,
            pltpu.SemaphoreType.DMA,
            pltpu.SemaphoreType.DMA,
            pltpu.SemaphoreType.DMA,
            pltpu.SemaphoreType.DMA,
            pltpu.SemaphoreType.DMA,
        ],
    )
    def k(xl_hbm, xr_hbm, src_hbm, dst_hbm, gxl_hbm, gxr_hbm,
          sfull, dfull, rl0, rl1, rl2, rr0, rr1, rr2, rlt, rrt,
          g0, g1, g2, w0, w1, w2):
        wid = lax.axis_index("s") * NC + lax.axis_index("c")
        base = wid * EPW
        slots = ((rl0, rr0, g0, w0), (rl1, rr1, g1, w1), (rl2, rr2, g2, w2))

        pltpu.sync_copy(src_hbm.at[pl.ds(base, EPW)], sfull)
        pltpu.sync_copy(dst_hbm.at[pl.ds(base, EPW)], dfull)

        def gather_start(c, rl, rr, gsem):
            o = c * KC
            pltpu.async_copy(xl_hbm.at[sfull.at[pl.ds(o, KC)]], rl, gsem)
            pltpu.async_copy(xr_hbm.at[dfull.at[pl.ds(o, KC)]], rr, gsem)

        def gather_wait(c, rl, rr, gsem):
            o = c * KC
            pltpu.make_async_copy(
                xl_hbm.at[sfull.at[pl.ds(o, KC)]], rl, gsem).wait()
            pltpu.make_async_copy(
                xr_hbm.at[dfull.at[pl.ds(o, KC)]], rr, gsem).wait()

        def wb_start(c, rl, rr, wsem):
            off = base + c * KC
            pltpu.async_copy(rl, gxl_hbm.at[pl.ds(off, KC)], wsem)
            pltpu.async_copy(rr, gxr_hbm.at[pl.ds(off, KC)], wsem)

        def wb_wait(c, rl, rr, wsem):
            off = base + c * KC
            pltpu.make_async_copy(rl, gxl_hbm.at[pl.ds(off, KC)], wsem).wait()
            pltpu.make_async_copy(rr, gxr_hbm.at[pl.ds(off, KC)], wsem).wait()

        for s in range(3):
            gather_start(s, slots[s][0], slots[s][1], slots[s][2])

        def addrows(rl, rr, nrows):
            @plsc.parallel_loop(0, nrows, unroll=4)
            def _(r):
                for cc in range(D // 16):
                    sl = pl.ds(cc * 16, 16)
                    rr[r, sl] = rl[r, sl] + rr[r, sl]

        def body(g, carry):
            for s in range(3):
                rl, rr, gsem, wsem = slots[s]
                c = 3 * g + s
                gather_wait(c, rl, rr, gsem)
                addrows(rl, rr, KC)
                wb_start(c, rl, rr, wsem)
                sp = (s + 2) % 3
                rlp, rrp, gsemp, wsemp = slots[sp]
                cr = c + 2

                @pl.when(jnp.logical_and(c >= 1, cr < NCH))
                def _():
                    wb_wait(c - 1, rlp, rrp, wsemp)
                    gather_start(cr, rlp, rrp, gsemp)

            return carry

        lax.fori_loop(0, NCH // 3, body, 0)
        for s in range(3):
            rl, rr, _, wsem = slots[s]
            wb_wait(NCH - 3 + s, rl, rr, wsem)

        toff = NCH * KC
        pltpu.async_copy(xl_hbm.at[sfull.at[pl.ds(toff, TAIL)]], rlt, g0)
        pltpu.async_copy(xr_hbm.at[dfull.at[pl.ds(toff, TAIL)]], rrt, g1)
        pltpu.make_async_copy(
            xl_hbm.at[sfull.at[pl.ds(toff, TAIL)]], rlt, g0).wait()
        pltpu.make_async_copy(
            xr_hbm.at[dfull.at[pl.ds(toff, TAIL)]], rrt, g1).wait()
        addrows(rlt, rrt, TAIL)
        pltpu.sync_copy(rlt, gxl_hbm.at[pl.ds(base + toff, TAIL)])
        pltpu.sync_copy(rrt, gxr_hbm.at[pl.ds(base + toff, TAIL)])

    return k(xl, xr, src, dst)


def _sc_scatter_add(vals, idx, gmax16=None):
    Dp = vals.shape[1]
    has_exp = gmax16 is not None

    scratch = [
        pltpu.VMEM((KC,), jnp.int32),
        pltpu.VMEM((KC,), jnp.int32),
        pltpu.VMEM((KC, Dp), jnp.float32),
        pltpu.VMEM((KC, Dp), jnp.float32),
        pltpu.VMEM((TAIL,), jnp.int32),
        pltpu.VMEM((TAIL, Dp), jnp.float32),
        pltpu.VMEM((NB, Dp), jnp.float32),
        pltpu.VMEM_SHARED((ACCN, Dp), jnp.float32),
        pltpu.SemaphoreType.DMA,
        pltpu.SemaphoreType.DMA,
        pltpu.SemaphoreType.DMA,
        pltpu.SemaphoreType.DMA,
    ]
    if has_exp:
        scratch.append(pltpu.VMEM((1, 16), jnp.float32))

    @functools.partial(
        pl.kernel,
        out_type=jax.ShapeDtypeStruct((NC * ACCN, Dp), jnp.float32),
        mesh=_mesh(),
        compiler_params=_SC_PARAMS,
        scratch_types=scratch,
    )
    def k(*refs):
        if has_exp:
            (vals_hbm, idx_hbm, gmax_hbm, out_hbm,
             idx0, idx1, rows0, rows1, idxt, rowst, zb_v, acc_sh,
             l0, l1, w0, w1, gm_v) = refs
        else:
            (vals_hbm, idx_hbm, out_hbm,
             idx0, idx1, rows0, rows1, idxt, rowst, zb_v, acc_sh,
             l0, l1, w0, w1) = refs
        cid = lax.axis_index("c")
        sid = lax.axis_index("s")
        wid = sid * NC + cid

        def zrow(i, carry):
            for cc in range(Dp // 16):
                zb_v[i, pl.ds(cc * 16, 16)] = jnp.zeros((16,), jnp.float32)
            return carry

        lax.fori_loop(0, NB, zrow, 0)
        for hop in range(NH):
            pltpu.sync_copy(zb_v, acc_sh.at[pl.ds(sid * NR + hop * NB, NB)])
        if has_exp:
            pltpu.sync_copy(gmax_hbm, gm_v)
        plsc.subcore_barrier()

        base = wid * EPW
        slots = ((idx0, rows0, l0, w0), (idx1, rows1, l1, w1))

        def load_start(c, idxv, rowsv, lsem):
            off = base + c * KC
            pltpu.async_copy(idx_hbm.at[pl.ds(off, KC)], idxv, lsem)
            pltpu.async_copy(vals_hbm.at[pl.ds(off, KC)], rowsv, lsem)

        for s in range(2):
            load_start(s, slots[s][0], slots[s][1], slots[s][2])

        def do_exp(rowsv, nrows):
            gm = gm_v[0, :]

            @plsc.parallel_loop(0, nrows, unroll=4)
            def _(r):
                rowsv[r, :] = jnp.exp(rowsv[r, :] - gm)

        def body(g, carry):
            for s in range(2):
                idxv, rowsv, lsem, wsem = slots[s]
                c = 2 * g + s
                off = base + c * KC
                pltpu.make_async_copy(
                    idx_hbm.at[pl.ds(off, KC)], idxv, lsem).wait()
                pltpu.make_async_copy(
                    vals_hbm.at[pl.ds(off, KC)], rowsv, lsem).wait()
                if has_exp:
                    do_exp(rowsv, KC)
                pltpu.async_copy(rowsv, acc_sh.at[idxv], wsem, add=True)
                cn = c + 2

                @pl.when(cn < NCH)
                def _():
                    pltpu.make_async_copy(rowsv, acc_sh.at[idxv], wsem).wait()
                    load_start(cn, idxv, rowsv, lsem)

            return carry

        lax.fori_loop(0, NCH // 2, body, 0)
        for s in range(2):
            idxv, rowsv, _, wsem = slots[s]
            pltpu.make_async_copy(rowsv, acc_sh.at[idxv], wsem).wait()

        toff = base + NCH * KC
        pltpu.sync_copy(idx_hbm.at[pl.ds(toff, TAIL)], idxt)
        pltpu.sync_copy(vals_hbm.at[pl.ds(toff, TAIL)], rowst)
        if has_exp:
            do_exp(rowst, TAIL)
        pltpu.sync_copy(rowst, acc_sh.at[idxt], add=True)
        plsc.subcore_barrier()

        for hop in range(NH):
            r0 = sid * NR + hop * NB
            pltpu.sync_copy(acc_sh.at[pl.ds(r0, NB)], zb_v)
            pltpu.sync_copy(zb_v, out_hbm.at[pl.ds(cid * ACCN + r0, NB)])

    if has_exp:
        return k(vals, idx, gmax16)
    return k(vals, idx)


def _node_proj(x, Wl, bl2, Wr, br2):
    NBLK = 2000

    def body(x_ref, wl_ref, bl_ref, wr_ref, br_ref, xl_ref, xr_ref):
        xb = x_ref[...]
        xl_ref[...] = jnp.dot(xb, wl_ref[...],
                              preferred_element_type=jnp.float32) + bl_ref[...]
        xr_ref[...] = jnp.dot(xb, wr_ref[...],
                              preferred_element_type=jnp.float32) + br_ref[...]

    return pl.pallas_call(
        body,
        grid=(N // NBLK,),
        in_specs=[
            pl.BlockSpec((NBLK, D), lambda i: (i, 0)),
            pl.BlockSpec((D, D), lambda i: (0, 0)),
            pl.BlockSpec((1, D), lambda i: (0, 0)),
            pl.BlockSpec((D, D), lambda i: (0, 0)),
            pl.BlockSpec((1, D), lambda i: (0, 0)),
        ],
        out_specs=[
            pl.BlockSpec((NBLK, D), lambda i: (i, 0)),
            pl.BlockSpec((NBLK, D), lambda i: (i, 0)),
        ],
        out_shape=[
            jax.ShapeDtypeStruct((N, D), jnp.float32),
            jax.ShapeDtypeStruct((N, D), jnp.float32),
        ],
    )(x, Wl, bl2, Wr, br2)


def _alpha(msum, edge_attr, We, att128, sel16):
    EB = 2000

    def body(ms_ref, ea_ref, we_ref, att_ref, sel_ref,
             alpha_ref, gmax_ref):
        eab = jnp.dot(ea_ref[...], we_ref[...],
                      preferred_element_type=jnp.float32)
        m = ms_ref[...] + eab
        m = jnp.where(m >= 0.0, m, 0.2 * m)
        t = m * att_ref[...]
        ab = jnp.dot(t, sel_ref[...], preferred_element_type=jnp.float32)
        alpha_ref[...] = ab
        bm = jnp.max(ab, axis=0, keepdims=True)

        @pl.when(pl.program_id(0) == 0)
        def _():
            gmax_ref[...] = bm

        @pl.when(pl.program_id(0) != 0)
        def _():
            gmax_ref[...] = jnp.maximum(gmax_ref[...], bm)

    return pl.pallas_call(
        body,
        grid=(E // EB,),
        in_specs=[
            pl.BlockSpec((EB, D), lambda i: (i, 0)),
            pl.BlockSpec((EB, DE), lambda i: (i, 0)),
            pl.BlockSpec((DE, D), lambda i: (0, 0)),
            pl.BlockSpec((1, D), lambda i: (0, 0)),
            pl.BlockSpec((D, 2 * H), lambda i: (0, 0)),
        ],
        out_specs=[
            pl.BlockSpec((EB, 2 * H), lambda i: (i, 0)),
            pl.BlockSpec((1, 2 * H), lambda i: (0, 0)),
        ],
        out_shape=[
            jax.ShapeDtypeStruct((E, 2 * H), jnp.float32),
            jax.ShapeDtypeStruct((1, 2 * H), jnp.float32),
        ],
    )(msum, edge_attr, We, att128, sel16)


def _wmsg(gxl, alpha16, gmax16, selT):
    EB = 2000

    def body(gxl_ref, al_ref, gm_ref, selT_ref, o_ref):
        a8 = jnp.exp(al_ref[:, :H] - gm_ref[:, :H])
        a128 = jnp.dot(a8, selT_ref[...], preferred_element_type=jnp.float32)
        o_ref[...] = gxl_ref[...] * a128

    return pl.pallas_call(
        body,
        grid=(E // EB,),
        in_specs=[
            pl.BlockSpec((EB, D), lambda i: (i, 0)),
            pl.BlockSpec((EB, 2 * H), lambda i: (i, 0)),
            pl.BlockSpec((1, 2 * H), lambda i: (0, 0)),
            pl.BlockSpec((H, D), lambda i: (0, 0)),
        ],
        out_specs=pl.BlockSpec((EB, D), lambda i: (i, 0)),
        out_shape=jax.ShapeDtypeStruct((E, D), jnp.float32),
    )(gxl, alpha16, gmax16, selT)


def _post(c2, d2, x, importance, bias2, Wg0, wgi, bg2, Wp, bp2, selT):
    NBLK = 2000

    def body(c0_ref, c1_ref, d0_ref, d1_ref, x_ref, imp_ref, bias_ref,
             wg0_ref, wgi_ref, bg_ref, wp_ref, bp_ref, selT_ref,
             out_ref, prop_ref):
        dn8 = d0_ref[:, :H] + d1_ref[:, :H]
        rec = 1.0 / (dn8 + 1e-16)
        rec128 = jnp.dot(rec, selT_ref[...], preferred_element_type=jnp.float32)
        conv = (c0_ref[...] + c1_ref[...]) * rec128 + bias_ref[...]
        logit = (jnp.dot(conv, wg0_ref[...],
                         preferred_element_type=jnp.float32)
                 + imp_ref[...] * wgi_ref[...] + bg_ref[...])
        gate = 1.0 / (1.0 + jnp.exp(-logit))
        out = gate * conv + (1.0 - gate) * x_ref[...]
        out_ref[...] = out
        prop_ref[...] = jnp.dot(out, wp_ref[...],
                                preferred_element_type=jnp.float32) + bp_ref[...]

    return pl.pallas_call(
        body,
        grid=(N // NBLK,),
        in_specs=[
            pl.BlockSpec((NBLK, D), lambda i: (i, 0)),
            pl.BlockSpec((NBLK, D), lambda i: (i, 0)),
            pl.BlockSpec((NBLK, 2 * H), lambda i: (i, 0)),
            pl.BlockSpec((NBLK, 2 * H), lambda i: (i, 0)),
            pl.BlockSpec((NBLK, D), lambda i: (i, 0)),
            pl.BlockSpec((NBLK, 1), lambda i: (i, 0)),
            pl.BlockSpec((1, D), lambda i: (0, 0)),
            pl.BlockSpec((D, D), lambda i: (0, 0)),
            pl.BlockSpec((1, D), lambda i: (0, 0)),
            pl.BlockSpec((1, D), lambda i: (0, 0)),
            pl.BlockSpec((D, 1), lambda i: (0, 0)),
            pl.BlockSpec((1, 1), lambda i: (0, 0)),
            pl.BlockSpec((H, D), lambda i: (0, 0)),
        ],
        out_specs=[
            pl.BlockSpec((NBLK, D), lambda i: (i, 0)),
            pl.BlockSpec((NBLK, 1), lambda i: (i, 0)),
        ],
        out_shape=[
            jax.ShapeDtypeStruct((N, D), jnp.float32),
            jax.ShapeDtypeStruct((N, 1), jnp.float32),
        ],
    )(c2[:N], c2[ACCN:ACCN + N], d2[:N], d2[ACCN:ACCN + N],
      x, importance, bias2, Wg0, wgi, bg2, Wp, bp2, selT)


def kernel(x, edge_index, edge_attr, importance, Wl, bl, Wr, br, We, att,
           bias, Wg, bg, Wp, bp):
    src = edge_index[0]
    dst = edge_index[1]
    att128 = att.reshape(1, H * C)
    sel = jnp.repeat(jnp.eye(H, dtype=jnp.float32), C, axis=0)
    sel16 = jnp.concatenate(
        [sel, jnp.zeros((H * C, H), jnp.float32)], axis=1)

    xl, xr = _node_proj(x, Wl, bl.reshape(1, D), Wr, br.reshape(1, D))
    gxl, msum = _sc_gather2(xl, xr, src, dst)
    alpha16, gmax16 = _alpha(msum, edge_attr, We, att128, sel16)
    wm = _wmsg(gxl, alpha16, gmax16, sel.T)
    conv2 = _sc_scatter_add(wm, dst)
    den2 = _sc_scatter_add(alpha16, dst, gmax16)
    out, prop = _post(conv2, den2, x, importance, bias.reshape(1, D),
                      Wg[:D], Wg[D].reshape(1, D), bg.reshape(1, D),
                      Wp, bp.reshape(1, 1), sel.T)
    return (out, prop)
